# baseline (speedup 1.0000x reference)
import functools

import jax
import jax.numpy as jnp
from jax import lax
from jax.experimental import pallas as pl
from jax.experimental.pallas import tpu as pltpu
from jax.experimental.pallas import tpu_sc as plsc

NC = 2
NS = 16
L = 16
NW = NC * NS

BATCH = 16384
SEQ = 48
HALF = 24
D = 64
RW = BATCH // NW
IW = RW * SEQ
G = 16
CROWS = G * SEQ
NCHUNK = RW // G


_mesh = plsc.VectorSubcoreMesh(
    core_axis_name="c", subcore_axis_name="s", num_cores=NC, num_subcores=NS
)


@functools.partial(
    pl.kernel,
    out_type=jax.ShapeDtypeStruct((NW, L), jnp.float32),
    mesh=_mesh,
    compiler_params=pltpu.CompilerParams(
        use_tc_tiling_on_sc=False, needs_layout_passes=False),
    scratch_types=[
        pltpu.VMEM((IW,), jnp.int32),
        pltpu.VMEM((CROWS, D), jnp.bfloat16),
        pltpu.VMEM((CROWS, D), jnp.bfloat16),
        pltpu.VMEM((L,), jnp.float32),
        pltpu.SemaphoreType.DMA,
        pltpu.SemaphoreType.DMA,
    ],
)
def _pair_loss(table_hbm, xf_hbm, out_hbm,
               xi_v, e0, e1, acc_v, s0, s1):
    wid = lax.axis_index("s") * NC + lax.axis_index("c")
    base = wid * IW

    pltpu.sync_copy(xf_hbm.at[pl.ds(base, IW)], xi_v)

    def start(g, e_buf, sem):
        pltpu.async_copy(table_hbm.at[xi_v.at[pl.ds(g * CROWS, CROWS)]],
                         e_buf, sem)

    def drain(e_buf, sem):
        pltpu.make_async_copy(table_hbm.at[pl.ds(0, CROWS)], e_buf, sem).wait()

    def compute(e_buf, accs):
        def brow(r, accs):
            out = list(accs)
            rb = r * SEQ
            for s in range(HALF):
                for j in range(2):
                    av = e_buf[rb + s, pl.ds(j * 2 * L, 2 * L)]
                    bv = e_buf[rb + s + HALF, pl.ds(j * 2 * L, 2 * L)]
                    dv = av - bv
                    d0, d1 = plsc.unpack(
                        dv, format=plsc.PackFormat.INTERLEAVED,
                        preferred_element_type=jnp.float32)
                    k = (s % 2) * 2 + j
                    out[k] = out[k] + (d0 * d0 + d1 * d1)
            return tuple(out)

        return lax.fori_loop(0, G, brow, accs)

    zeros = jnp.zeros((L,), jnp.float32)
    accs = (zeros, zeros, zeros, zeros)

    start(0, e0, s0)

    def body(h, accs):
        g = 2 * h
        start(g + 1, e1, s1)
        drain(e0, s0)
        accs = compute(e0, accs)
        start(g + 2, e0, s0)
        drain(e1, s1)
        return compute(e1, accs)

    accs = lax.fori_loop(0, NCHUNK // 2 - 1, body, accs)

    start(NCHUNK - 1, e1, s1)
    drain(e0, s0)
    accs = compute(e0, accs)
    drain(e1, s1)
    accs = compute(e1, accs)

    acc_v[...] = (accs[0] + accs[1]) + (accs[2] + accs[3])
    pltpu.sync_copy(acc_v, out_hbm.at[wid])


def kernel(x, embd_size, table):
    partials = _pair_loss(table.astype(jnp.bfloat16), x.reshape(-1))
    return jnp.sum(partials)


# speedup vs baseline: 1.0132x; 1.0132x over previous
import functools

import jax
import jax.numpy as jnp
from jax import lax
from jax.experimental import pallas as pl
from jax.experimental.pallas import tpu as pltpu
from jax.experimental.pallas import tpu_sc as plsc

NC = 2
NS = 16
L = 16
NW = NC * NS

BATCH = 16384
SEQ = 48
HALF = 24
D = 64
RW = BATCH // NW
IW = RW * SEQ
G = 16
CROWS = G * SEQ
NCHUNK = RW // G


_mesh = plsc.VectorSubcoreMesh(
    core_axis_name="c", subcore_axis_name="s", num_cores=NC, num_subcores=NS
)


@functools.partial(
    pl.kernel,
    out_type=jax.ShapeDtypeStruct((NW, L), jnp.float32),
    mesh=_mesh,
    compiler_params=pltpu.CompilerParams(
        use_tc_tiling_on_sc=False, needs_layout_passes=False),
    scratch_types=[
        pltpu.VMEM((IW,), jnp.int32),
        pltpu.VMEM((CROWS, D), jnp.bfloat16),
        pltpu.VMEM((CROWS, D), jnp.bfloat16),
        pltpu.VMEM((L,), jnp.float32),
        pltpu.SemaphoreType.DMA,
        pltpu.SemaphoreType.DMA,
    ],
)
def _pair_loss(table_hbm, xf_hbm, out_hbm,
               xi_v, e0, e1, acc_v, s0, s1):
    wid = lax.axis_index("s") * NC + lax.axis_index("c")
    base = wid * IW

    pltpu.sync_copy(xf_hbm.at[pl.ds(base, IW)], xi_v)

    def start(g, e_buf, sem):
        pltpu.async_copy(table_hbm.at[xi_v.at[pl.ds(g * CROWS, CROWS)]],
                         e_buf, sem)

    def drain(e_buf, sem):
        pltpu.make_async_copy(table_hbm.at[pl.ds(0, CROWS)], e_buf, sem).wait()

    def compute(e_buf, accs):
        def brow(r, accs):
            out = list(accs)
            rb = r * SEQ
            for s in range(HALF):
                for j in range(2):
                    av = e_buf[rb + s, pl.ds(j * 2 * L, 2 * L)]
                    bv = e_buf[rb + s + HALF, pl.ds(j * 2 * L, 2 * L)]
                    dv = av - bv
                    d0, d1 = plsc.unpack(
                        dv, format=plsc.PackFormat.INTERLEAVED,
                        preferred_element_type=jnp.float32)
                    out[2 * j] = out[2 * j] + d0 * d0
                    out[2 * j + 1] = out[2 * j + 1] + d1 * d1
            return tuple(out)

        return lax.fori_loop(0, G, brow, accs)

    zeros = jnp.zeros((L,), jnp.float32)
    accs = (zeros, zeros, zeros, zeros)

    start(0, e0, s0)

    def body(h, accs):
        g = 2 * h
        start(g + 1, e1, s1)
        drain(e0, s0)
        accs = compute(e0, accs)
        start(g + 2, e0, s0)
        drain(e1, s1)
        return compute(e1, accs)

    accs = lax.fori_loop(0, NCHUNK // 2 - 1, body, accs)

    start(NCHUNK - 1, e1, s1)
    drain(e0, s0)
    accs = compute(e0, accs)
    drain(e1, s1)
    accs = compute(e1, accs)

    acc_v[...] = (accs[0] + accs[1]) + (accs[2] + accs[3])
    pltpu.sync_copy(acc_v, out_hbm.at[wid])


def kernel(x, embd_size, table):
    partials = _pair_loss(table.astype(jnp.bfloat16), x.reshape(-1))
    return jnp.sum(partials)
